# baseline (speedup 1.0000x reference)
import functools

import jax
import jax.numpy as jnp
from jax import lax
from jax.experimental import pallas as pl
from jax.experimental.pallas import tpu as pltpu
from jax.experimental.pallas import tpu_sc as plsc

_EMB = 64
_LSP = 208
_LAP = 32
_TOT = _LSP + _LAP
_NCHUNK = 3
_CHUNK = _TOT // _NCHUNK
_NC, _NS = 2, 16
_NW = _NC * _NS


def _sc_embed_means(tok3, emb_table):
    B = tok3.shape[0]
    per_w = B // _NW
    n_state_vregs = _LSP // 16
    mesh = plsc.VectorSubcoreMesh(
        core_axis_name="c", subcore_axis_name="s",
        num_cores=_NC, num_subcores=_NS)

    @functools.partial(
        pl.kernel,
        out_type=(jax.ShapeDtypeStruct((B, _EMB), jnp.float32),
                  jax.ShapeDtypeStruct((B, _EMB), jnp.float32)),
        mesh=mesh,
        compiler_params=pltpu.CompilerParams(
            needs_layout_passes=False, use_tc_tiling_on_sc=False),
        scratch_types=[
            pltpu.VMEM((2, _NCHUNK, _CHUNK), jnp.int32),
            pltpu.VMEM((2, _TOT, _EMB), jnp.float32),
            pltpu.VMEM((per_w, _EMB), jnp.float32),
            pltpu.VMEM((per_w, _EMB), jnp.float32),
            pltpu.SemaphoreType.DMA,
            pltpu.SemaphoreType.DMA,
        ],
    )
    def k(tok_hbm, e_hbm, out_s_hbm, out_a_hbm,
          idx_v, rows_v, os_v, oa_v, sem0, sem1):
        wid = lax.axis_index("s") * _NC + lax.axis_index("c")
        base = wid * per_w
        sems = (sem0, sem1)

        def issue(lb, buf):
            idx3 = idx_v.at[buf]
            pltpu.sync_copy(tok_hbm.at[base + lb], idx3)
            for c in range(_NCHUNK):
                pltpu.async_copy(
                    e_hbm.at[idx3.at[c]],
                    rows_v.at[buf, pl.ds(c * _CHUNK, _CHUNK)],
                    sems[buf])

        def consume(lb, buf):
            idx3 = idx_v.at[buf]
            for c in range(_NCHUNK):
                pltpu.make_async_copy(
                    e_hbm.at[idx3.at[c]],
                    rows_v.at[buf, pl.ds(c * _CHUNK, _CHUNK)],
                    sems[buf]).wait()
            cs = jnp.zeros((16,), jnp.int32)
            ca = jnp.zeros((16,), jnp.int32)
            for kc in range(_TOT // 16):
                t = idx3[kc // 5, pl.ds((kc % 5) * 16, 16)]
                pc = plsc.all_reduce_population_count(t != 0)
                if kc < n_state_vregs:
                    cs = cs + pc
                else:
                    ca = ca + pc
            inv_s = 1.0 / jnp.maximum(cs.astype(jnp.float32), 1.0)
            inv_a = 1.0 / jnp.maximum(ca.astype(jnp.float32), 1.0)

            def rbody(r, acc):
                return tuple(
                    acc[j] + rows_v[buf, r, pl.ds(16 * j, 16)]
                    for j in range(_EMB // 16))

            z4 = tuple(jnp.zeros((16,), jnp.float32)
                       for _ in range(_EMB // 16))
            acc_s = lax.fori_loop(0, _LSP, rbody, z4, unroll=4)
            acc_a = lax.fori_loop(_LSP, _TOT, rbody, z4, unroll=4)
            for j in range(_EMB // 16):
                os_v[lb, pl.ds(16 * j, 16)] = acc_s[j] * inv_s
                oa_v[lb, pl.ds(16 * j, 16)] = acc_a[j] * inv_a

        issue(0, 0)
        nit = per_w // 2

        def body(i, carry):
            b0 = 2 * i
            issue(b0 + 1, 1)
            consume(b0, 0)

            @pl.when(i < nit - 1)
            def _():
                issue(b0 + 2, 0)

            consume(b0 + 1, 1)
            return carry

        lax.fori_loop(0, nit, body, 0)
        pltpu.sync_copy(os_v, out_s_hbm.at[pl.ds(base, per_w)])
        pltpu.sync_copy(oa_v, out_a_hbm.at[pl.ds(base, per_w)])

    return k(tok3, emb_table)


def _tc_mlp(ms, ma, w1t, b1, w2t, b2, w3a, w3b, b3, w4, b4):
    B, H = ms.shape[0], w1t.shape[1]
    BS = 2048

    def body(ms_ref, ma_ref, w1_ref, b1_ref, w2_ref, b2_ref,
             w3a_ref, w3b_ref, b3_ref, w4_ref, b4_ref, out_ref):
        hs = jnp.tanh(jnp.dot(ms_ref[...], w1_ref[...],
                              preferred_element_type=jnp.float32) + b1_ref[...])
        ha = jnp.tanh(jnp.dot(ma_ref[...], w2_ref[...],
                              preferred_element_type=jnp.float32) + b2_ref[...])
        hs = jnp.maximum(hs, 0.0)
        ha = jnp.maximum(ha, 0.0)
        h = (jnp.dot(hs, w3a_ref[...], preferred_element_type=jnp.float32)
             + jnp.dot(ha, w3b_ref[...], preferred_element_type=jnp.float32)
             + b3_ref[...])
        h = jnp.maximum(h, 0.0)
        out_ref[...] = (jnp.sum(h * w4_ref[...], axis=1, keepdims=True)
                        + b4_ref[...])

    return pl.pallas_call(
        body,
        grid=(B // BS,),
        in_specs=[
            pl.BlockSpec((BS, _EMB), lambda i: (i, 0)),
            pl.BlockSpec((BS, _EMB), lambda i: (i, 0)),
            pl.BlockSpec((_EMB, H), lambda i: (0, 0)),
            pl.BlockSpec((1, H), lambda i: (0, 0)),
            pl.BlockSpec((_EMB, H), lambda i: (0, 0)),
            pl.BlockSpec((1, H), lambda i: (0, 0)),
            pl.BlockSpec((H, H), lambda i: (0, 0)),
            pl.BlockSpec((H, H), lambda i: (0, 0)),
            pl.BlockSpec((1, H), lambda i: (0, 0)),
            pl.BlockSpec((1, H), lambda i: (0, 0)),
            pl.BlockSpec((1, 1), lambda i: (0, 0)),
        ],
        out_specs=pl.BlockSpec((BS, 1), lambda i: (i, 0)),
        out_shape=jax.ShapeDtypeStruct((B, 1), jnp.float32),
    )(ms, ma, w1t, b1, w2t, b2, w3a, w3b, b3, w4, b4)


def kernel(state_tokens, state_lengths, action_tokens, action_lengths,
           E, W1, b1, W2, b2, W3, b3, W4, b4):
    del state_lengths, action_lengths
    B = state_tokens.shape[0]
    H = W1.shape[0]
    st = state_tokens.astype(jnp.int32)
    at = action_tokens.astype(jnp.int32)
    tok = jnp.concatenate([
        jnp.pad(st, ((0, 0), (0, _LSP - st.shape[1]))),
        jnp.pad(at, ((0, 0), (0, _LAP - at.shape[1]))),
    ], axis=1).reshape(B, _NCHUNK, _CHUNK)
    ms, ma = _sc_embed_means(tok, E)
    w3t = W3.T
    return _tc_mlp(ms, ma,
                   W1.T, b1.reshape(1, H),
                   W2.T, b2.reshape(1, H),
                   w3t[:H], w3t[H:], b3.reshape(1, H),
                   W4, b4.reshape(1, 1))
